# Initial kernel scaffold; baseline (speedup 1.0000x reference)
import jax
import jax.numpy as jnp
from jax.experimental import pallas as pl


def kernel(x, table):
    raise NotImplementedError("write your pallas kernel here")


# speedup vs baseline: 1.0433x; 1.0433x over previous
import functools

import jax
import jax.numpy as jnp
from jax import lax
from jax.experimental import pallas as pl
from jax.experimental.pallas import tpu as pltpu
from jax.experimental.pallas import tpu_sc as plsc

_LANES = 128
_D = 65
_DP = 128
_P1_CHUNK = 200
_P2_G = 2

_COL_OFFS = (0, 16, 32, 48, 49)


@functools.cache
def _make_phase1(n_rows):
    info = plsc.get_sparse_core_info()
    nw = info.num_cores * info.num_subcores
    n_chunks = n_rows // _P1_CHUNK
    per_w = -(-n_chunks // nw)
    mesh = plsc.VectorSubcoreMesh(core_axis_name="c", subcore_axis_name="s")

    @functools.partial(
        pl.kernel,
        mesh=mesh,
        out_type=jax.ShapeDtypeStruct((n_rows, _DP), jnp.float32),
        scratch_types=[
            pltpu.VMEM((_P1_CHUNK, _D), jnp.float32),
            pltpu.VMEM((_P1_CHUNK, _DP), jnp.float32),
        ],
    )
    def phase1(table_hbm, padded_hbm, buf65, buf128):
        wid = lax.axis_index("s") * info.num_cores + lax.axis_index("c")

        def body(k, carry):
            g = wid + k * nw

            @pl.when(g < n_chunks)
            def _():
                r0 = g * _P1_CHUNK
                pltpu.sync_copy(table_hbm.at[pl.ds(r0, _P1_CHUNK)], buf65)

                def row(i, c):
                    for j in _COL_OFFS:
                        buf128[i, pl.ds(j, 16)] = buf65[i, pl.ds(j, 16)]
                    return c

                lax.fori_loop(0, _P1_CHUNK, row, 0)
                pltpu.sync_copy(buf128, padded_hbm.at[pl.ds(r0, _P1_CHUNK)])

            return carry

        lax.fori_loop(0, per_w, body, 0)

    return phase1


@functools.cache
def _make_phase2(total, n_rows):
    info = plsc.get_sparse_core_info()
    nw = info.num_cores * info.num_subcores
    rows128 = total // _LANES
    per_w = rows128 // nw
    n_chunks = per_w // _P2_G
    chunk_rows = _P2_G * _LANES
    mesh = plsc.VectorSubcoreMesh(core_axis_name="c", subcore_axis_name="s")

    @functools.partial(
        pl.kernel,
        mesh=mesh,
        out_type=jax.ShapeDtypeStruct((total, _D), jnp.float32),
        scratch_types=[
            pltpu.VMEM((_P2_G, _LANES), jnp.int32),
            pltpu.VMEM((chunk_rows, _DP), jnp.float32),
            pltpu.VMEM((chunk_rows, _D), jnp.float32),
            pltpu.SemaphoreType.DMA,
        ],
    )
    def phase2(idx_hbm, padded_hbm, out_hbm, idx_v, rows128_v, rows65_v, sem):
        wid = lax.axis_index("s") * info.num_cores + lax.axis_index("c")
        base = wid * per_w

        def body(c, carry):
            g0 = base + c * _P2_G
            pltpu.sync_copy(idx_hbm.at[pl.ds(g0, _P2_G)], idx_v)
            cps = [
                pltpu.async_copy(
                    padded_hbm.at[idx_v.at[j]],
                    rows128_v.at[pl.ds(j * _LANES, _LANES)],
                    sem,
                )
                for j in range(_P2_G)
            ]
            for cp in cps:
                cp.wait()

            def row(i, cc):
                for j in _COL_OFFS:
                    rows65_v[i, pl.ds(j, 16)] = rows128_v[i, pl.ds(j, 16)]
                return cc

            lax.fori_loop(0, chunk_rows, row, 0)
            pltpu.sync_copy(rows65_v, out_hbm.at[pl.ds(g0 * _LANES, chunk_rows)])
            return carry

        lax.fori_loop(0, n_chunks, body, 0)

    return phase2


def kernel(x, table):
    b, h = x.shape
    total = b * h
    n_rows = table.shape[0]
    idx = x.reshape(total // _LANES, _LANES).astype(jnp.int32)
    padded = _make_phase1(n_rows)(table)
    out = _make_phase2(total, n_rows)(idx, padded)
    return out.reshape(b, h, table.shape[1])


# speedup vs baseline: 1.0724x; 1.0279x over previous
import functools

import jax
import jax.numpy as jnp
from jax import lax
from jax.experimental import pallas as pl
from jax.experimental.pallas import tpu as pltpu
from jax.experimental.pallas import tpu_sc as plsc

_LANES = 128
_D = 65
_DP = 128
_P1_CHUNK = 200
_P2_G = 2

_COL_OFFS = (0, 16, 32, 48, 49)


@functools.cache
def _make_phase1(n_rows):
    info = plsc.get_sparse_core_info()
    nw = info.num_cores * info.num_subcores
    n_chunks = n_rows // _P1_CHUNK
    per_w = -(-n_chunks // nw)
    mesh = plsc.VectorSubcoreMesh(core_axis_name="c", subcore_axis_name="s")

    @functools.partial(
        pl.kernel,
        mesh=mesh,
        out_type=jax.ShapeDtypeStruct((n_rows, _DP), jnp.float32),
        scratch_types=[
            pltpu.VMEM((_P1_CHUNK, _D), jnp.float32),
            pltpu.VMEM((_P1_CHUNK, _DP), jnp.float32),
        ],
    )
    def phase1(table_hbm, padded_hbm, buf65, buf128):
        wid = lax.axis_index("s") * info.num_cores + lax.axis_index("c")

        def body(k, carry):
            g = wid + k * nw

            @pl.when(g < n_chunks)
            def _():
                r0 = g * _P1_CHUNK
                pltpu.sync_copy(table_hbm.at[pl.ds(r0, _P1_CHUNK)], buf65)

                def row(i, c):
                    for j in _COL_OFFS:
                        buf128[i, pl.ds(j, 16)] = buf65[i, pl.ds(j, 16)]
                    return c

                lax.fori_loop(0, _P1_CHUNK, row, 0)
                pltpu.sync_copy(buf128, padded_hbm.at[pl.ds(r0, _P1_CHUNK)])

            return carry

        lax.fori_loop(0, per_w, body, 0)

    return phase1


@functools.cache
def _make_phase2(b, h, n_rows):
    info = plsc.get_sparse_core_info()
    nw = info.num_cores * info.num_subcores
    per_w = b // nw
    n_chunks = per_w // _P2_G
    chunk_rows = _P2_G * h
    total = b * h
    h_full, h_rem = divmod(h, _LANES)
    mesh = plsc.VectorSubcoreMesh(core_axis_name="c", subcore_axis_name="s")

    @functools.partial(
        pl.kernel,
        mesh=mesh,
        out_type=jax.ShapeDtypeStruct((total, _D), jnp.float32),
        scratch_types=[
            pltpu.VMEM((_P2_G, h), jnp.int32),
            pltpu.VMEM((chunk_rows, _DP), jnp.float32),
            pltpu.VMEM((chunk_rows, _D), jnp.float32),
            pltpu.SemaphoreType.DMA,
        ],
    )
    def phase2(idx_hbm, padded_hbm, out_hbm, idx_v, rows128_v, rows65_v, sem):
        wid = lax.axis_index("s") * info.num_cores + lax.axis_index("c")
        base = wid * per_w

        def body(c, carry):
            b0 = base + c * _P2_G
            pltpu.sync_copy(idx_hbm.at[pl.ds(b0, _P2_G)], idx_v)
            cps = []
            for j in range(_P2_G):
                for k in range(h_full):
                    cps.append(pltpu.async_copy(
                        padded_hbm.at[idx_v.at[j, pl.ds(k * _LANES, _LANES)]],
                        rows128_v.at[pl.ds(j * h + k * _LANES, _LANES)],
                        sem,
                    ))
                if h_rem:
                    cps.append(pltpu.async_copy(
                        padded_hbm.at[idx_v.at[j, pl.ds(h_full * _LANES, h_rem)]],
                        rows128_v.at[pl.ds(j * h + h_full * _LANES, h_rem)],
                        sem,
                    ))
            for cp in cps:
                cp.wait()

            def row(i, cc):
                for j in _COL_OFFS:
                    rows65_v[i, pl.ds(j, 16)] = rows128_v[i, pl.ds(j, 16)]
                return cc

            lax.fori_loop(0, chunk_rows, row, 0)
            pltpu.sync_copy(rows65_v, out_hbm.at[pl.ds(b0 * h, chunk_rows)])
            return carry

        lax.fori_loop(0, n_chunks, body, 0)

    return phase2


def kernel(x, table):
    b, h = x.shape
    n_rows = table.shape[0]
    idx = x.astype(jnp.int32)
    padded = _make_phase1(n_rows)(table)
    out = _make_phase2(b, h, n_rows)(idx, padded)
    return out.reshape(b, h, table.shape[1])
